# baseline (speedup 1.0000x reference)
import functools

import jax
import jax.numpy as jnp
from jax import lax
from jax.experimental import pallas as pl
from jax.experimental.pallas import tpu as pltpu
from jax.experimental.pallas import tpu_sc as plsc

_BATCH = 16384
_DIM = 32
_LANES = 128
_NUM_CORES = 2
_NUM_SUBCORES = 16
_NUM_WORKERS = _NUM_CORES * _NUM_SUBCORES
_B_PER_W = _BATCH // _NUM_WORKERS
_HALF = 8
_GROUP = 16
_NGROUP = _B_PER_W // _GROUP
_TC_PER_W = _B_PER_W // _LANES
_NTC = _BATCH // _LANES


def _gather_body(idx_hbm, tableT_hbm, out_hbm, idx_v, q_v, r_v, bp, bq, br,
                 out_v, sp, sq, sr):
    wid = lax.axis_index("s") * _NUM_CORES + lax.axis_index("c")
    base = wid * _B_PER_W
    pltpu.sync_copy(idx_hbm.at[pl.ds(base, _B_PER_W)], idx_v)

    @plsc.parallel_loop(0, _B_PER_W, 16)
    def _(i):
        v = idx_v[pl.ds(i, 16)]
        q_v[pl.ds(i, 16)] = v & jnp.int32(~(_LANES - 1))
        r_v[pl.ds(i, 16)] = v & jnp.int32(_LANES - 1)

    lanes = lax.iota(jnp.int32, 16)
    bvec = lanes & jnp.int32(_HALF - 1)
    lo_mask = lanes < jnp.int32(_HALF)

    def issue8(k0, buf, sem):
        qv = plsc.load_gather(q_v, [k0 + lanes])
        for j in range(_HALF):
            pltpu.make_async_copy(
                tableT_hbm.at[
                    :, pl.ds(pl.multiple_of(qv[j], _LANES), _LANES)],
                buf.at[j],
                sem,
            ).start()

    def wait8(buf, sem):
        for j in range(_HALF):
            pltpu.make_async_copy(
                tableT_hbm.at[:, pl.ds(0, _LANES)], buf.at[j], sem
            ).wait()

    def extract16(k0, buf_a, buf_b):
        rv = plsc.load_gather(r_v, [k0 + lanes])
        base0 = (k0 // _LANES) * 1024 + k0 % _LANES
        for c in range(_DIM):
            csplat = jnp.full((16,), c, jnp.int32)
            va = plsc.load_gather(buf_a, [bvec, csplat, rv])
            vb = plsc.load_gather(buf_b, [bvec, csplat, rv])
            pos = base0 + ((c // 8) * _TC_PER_W) * 1024 + (c % 8) * _LANES
            out_v[pl.ds(pos, 16)] = jnp.where(lo_mask, va, vb)

    def step(k0, a, sa, b, sb, c, sc, prefetch):
        if prefetch:
            issue8(k0 + _GROUP, c, sc)
        wait8(a, sa)
        wait8(b, sb)
        extract16(k0, a, b)
        if prefetch:
            issue8(k0 + _GROUP + _HALF, a, sa)

    issue8(0, bp, sp)
    issue8(_HALF, bq, sq)

    def loop_body(t, carry):
        k0 = t * (3 * _GROUP)
        step(k0, bp, sp, bq, sq, br, sr, True)
        step(k0 + _GROUP, br, sr, bp, sp, bq, sq, True)
        step(k0 + 2 * _GROUP, bq, sq, br, sr, bp, sp, True)
        return carry

    lax.fori_loop(0, _NGROUP // 3, loop_body, 0)
    step((_NGROUP - 2) * _GROUP, bp, sp, bq, sq, br, sr, True)
    step((_NGROUP - 1) * _GROUP, br, sr, bp, sp, bq, sq, False)

    for tr in range(_DIM // 8):
        for tcl in range(_TC_PER_W):
            tc = wid * _TC_PER_W + tcl
            pltpu.sync_copy(
                out_v.at[pl.ds((tr * _TC_PER_W + tcl) * 1024, 1024)],
                out_hbm.at[pl.ds((tr * _NTC + tc) * 1024, 1024)],
            )


@jax.jit
def kernel(x, embedding_publisher):
    idx = x[:, 0].astype(jnp.int32)
    tableT = embedding_publisher.T
    mesh = plsc.VectorSubcoreMesh(core_axis_name="c", subcore_axis_name="s")
    run = functools.partial(
        pl.kernel,
        mesh=mesh,
        out_type=jax.ShapeDtypeStruct((_BATCH * _DIM,), jnp.float32),
        scratch_types=[
            pltpu.VMEM((_B_PER_W,), jnp.int32),
            pltpu.VMEM((_B_PER_W,), jnp.int32),
            pltpu.VMEM((_B_PER_W,), jnp.int32),
            pltpu.VMEM((_HALF, _DIM, _LANES), jnp.float32),
            pltpu.VMEM((_HALF, _DIM, _LANES), jnp.float32),
            pltpu.VMEM((_HALF, _DIM, _LANES), jnp.float32),
            pltpu.VMEM((_B_PER_W * _DIM,), jnp.float32),
            pltpu.SemaphoreType.DMA,
            pltpu.SemaphoreType.DMA,
            pltpu.SemaphoreType.DMA,
        ],
        compiler_params=pltpu.CompilerParams(needs_layout_passes=False),
    )(_gather_body)
    out_flat = run(idx, tableT)
    out = (out_flat.reshape(_DIM // 8, _NTC, 8, _LANES)
           .transpose(0, 2, 1, 3)
           .reshape(_DIM, _BATCH)
           .T)
    return out


# speedup vs baseline: 1.0320x; 1.0320x over previous
import functools

import jax
import jax.numpy as jnp
from jax import lax
from jax.experimental import pallas as pl
from jax.experimental.pallas import tpu as pltpu
from jax.experimental.pallas import tpu_sc as plsc

_BATCH = 16384
_DIM = 32
_LANES = 128
_NUM_CORES = 2
_NUM_SUBCORES = 16
_NUM_WORKERS = _NUM_CORES * _NUM_SUBCORES
_B_PER_W = _BATCH // _NUM_WORKERS
_HALF = 8
_GROUP = 16
_NHALF = _B_PER_W // _HALF
_NGROUP = _B_PER_W // _GROUP
_SLOTS = 24
_TC_PER_W = _B_PER_W // _LANES
_NTC = _BATCH // _LANES


def _gather_body(idx_hbm, tableT_hbm, out_hbm, idx_v, q_v, r_v, blocks_v,
                 out_v, s0, s1, s2, s3):
    wid = lax.axis_index("s") * _NUM_CORES + lax.axis_index("c")
    base = wid * _B_PER_W
    pltpu.sync_copy(idx_hbm.at[pl.ds(base, _B_PER_W)], idx_v)

    @plsc.parallel_loop(0, _B_PER_W, 16)
    def _(i):
        v = idx_v[pl.ds(i, 16)]
        q_v[pl.ds(i, 16)] = v & jnp.int32(~(_LANES - 1))
        r_v[pl.ds(i, 16)] = v & jnp.int32(_LANES - 1)

    lanes = lax.iota(jnp.int32, 16)

    def issue_half(h, sem):
        qv = plsc.load_gather(q_v, [h * _HALF + lanes])
        slot0 = (h % 3) * _HALF
        for j in range(_HALF):
            pltpu.make_async_copy(
                tableT_hbm.at[
                    :, pl.ds(pl.multiple_of(qv[j], _LANES), _LANES)],
                blocks_v.at[slot0 + j],
                sem,
            ).start()

    def wait_half(sem):
        for j in range(_HALF):
            pltpu.make_async_copy(
                tableT_hbm.at[:, pl.ds(0, _LANES)], blocks_v.at[j], sem
            ).wait()

    def extract_group(g):
        k0 = g * _GROUP
        rv = r_v[pl.ds(k0, 16)]
        hvec = 2 * g + lax.shift_right_logical(lanes, 3)
        slotvec = lax.rem(hvec, 3) * _HALF + (lanes & jnp.int32(_HALF - 1))
        tcl = k0 // _LANES
        kin = k0 % _LANES
        for c in range(_DIM):
            vals = plsc.load_gather(
                blocks_v, [slotvec, jnp.full((16,), c, jnp.int32), rv])
            pos = ((c // 8) * _TC_PER_W + tcl) * 1024 + (c % 8) * 128 + kin
            out_v[pl.ds(pos, 16)] = vals

    def group_step(g, sa, sb, sc_, sd):
        h2 = 2 * g + 2
        h3 = 2 * g + 3

        @pl.when(h2 < _NHALF)
        def _():
            issue_half(h2, sc_)

        wait_half(sa)
        wait_half(sb)
        extract_group(g)

        @pl.when(h3 < _NHALF)
        def _():
            issue_half(h3, sd)

    issue_half(0, s0)
    issue_half(1, s1)

    def loop_body(t, carry):
        group_step(2 * t, s0, s1, s2, s3)
        group_step(2 * t + 1, s2, s3, s0, s1)
        return carry

    lax.fori_loop(0, _NGROUP // 2, loop_body, 0)

    for tr in range(_DIM // 8):
        for tcl in range(_TC_PER_W):
            tc = wid * _TC_PER_W + tcl
            pltpu.sync_copy(
                out_v.at[pl.ds((tr * _TC_PER_W + tcl) * 1024, 1024)],
                out_hbm.at[pl.ds((tr * _NTC + tc) * 1024, 1024)],
            )


@jax.jit
def kernel(x, embedding_publisher):
    idx = x[:, 0].astype(jnp.int32)
    tableT = embedding_publisher.T
    mesh = plsc.VectorSubcoreMesh(core_axis_name="c", subcore_axis_name="s")
    run = functools.partial(
        pl.kernel,
        mesh=mesh,
        out_type=jax.ShapeDtypeStruct((_BATCH * _DIM,), jnp.float32),
        scratch_types=[
            pltpu.VMEM((_B_PER_W,), jnp.int32),
            pltpu.VMEM((_B_PER_W,), jnp.int32),
            pltpu.VMEM((_B_PER_W,), jnp.int32),
            pltpu.VMEM((_SLOTS, _DIM, _LANES), jnp.float32),
            pltpu.VMEM((_B_PER_W * _DIM,), jnp.float32),
            pltpu.SemaphoreType.DMA,
            pltpu.SemaphoreType.DMA,
            pltpu.SemaphoreType.DMA,
            pltpu.SemaphoreType.DMA,
        ],
        compiler_params=pltpu.CompilerParams(needs_layout_passes=False),
    )(_gather_body)
    out_flat = run(idx, tableT)
    out = (out_flat.reshape(_DIM // 8, _NTC, 8, _LANES)
           .transpose(0, 2, 1, 3)
           .reshape(_DIM, _BATCH)
           .T)
    return out


# speedup vs baseline: 1.0571x; 1.0243x over previous
import functools

import jax
import jax.numpy as jnp
from jax import lax
from jax.experimental import pallas as pl
from jax.experimental.pallas import tpu as pltpu
from jax.experimental.pallas import tpu_sc as plsc

_BATCH = 16384
_DIM = 32
_LANES = 128
_NUM_CORES = 2
_NUM_SUBCORES = 16
_NUM_WORKERS = _NUM_CORES * _NUM_SUBCORES
_B_PER_W = _BATCH // _NUM_WORKERS
_CHUNK = 16
_NCHUNK = _B_PER_W // _CHUNK
_TC_PER_W = _B_PER_W // _LANES
_NTC = _BATCH // _LANES


def _gather_body(idx_hbm, tableT_hbm, out_hbm, idx_v, q_v, r_v, blocks_v,
                 out_v, sem):
    wid = lax.axis_index("s") * _NUM_CORES + lax.axis_index("c")
    base = wid * _B_PER_W
    pltpu.sync_copy(idx_hbm.at[pl.ds(base, _B_PER_W)], idx_v)

    @plsc.parallel_loop(0, _B_PER_W, 16)
    def _(i):
        v = idx_v[pl.ds(i, 16)]
        q_v[pl.ds(i, 16)] = v & jnp.int32(~(_LANES - 1))
        r_v[pl.ds(i, 16)] = v & jnp.int32(_LANES - 1)

    lanes = lax.iota(jnp.int32, 16)

    def chunk_body(g, carry):
        k0 = g * _CHUNK
        qv = q_v[pl.ds(k0, 16)]
        copies = []
        for j in range(_CHUNK):
            q = pl.multiple_of(qv[j], _LANES)
            for b in range(_DIM // 8):
                copies.append(pltpu.make_async_copy(
                    tableT_hbm.at[pl.ds(8 * b, 8), pl.ds(q, _LANES)],
                    blocks_v.at[j, pl.ds(8 * b, 8), :],
                    sem,
                ))
        for c in copies:
            c.start()
        for c in copies:
            c.wait()

        rv = r_v[pl.ds(k0, 16)]
        tcl = k0 // _LANES
        kin = k0 % _LANES
        for c in range(_DIM):
            vals = plsc.load_gather(
                blocks_v, [lanes, jnp.full((16,), c, jnp.int32), rv])
            pos = ((c // 8) * _TC_PER_W + tcl) * 1024 + (c % 8) * 128 + kin
            out_v[pl.ds(pos, 16)] = vals
        return carry

    lax.fori_loop(0, _NCHUNK, chunk_body, 0)

    for tr in range(_DIM // 8):
        for tcl in range(_TC_PER_W):
            tc = wid * _TC_PER_W + tcl
            pltpu.sync_copy(
                out_v.at[pl.ds((tr * _TC_PER_W + tcl) * 1024, 1024)],
                out_hbm.at[pl.ds((tr * _NTC + tc) * 1024, 1024)],
            )


@jax.jit
def kernel(x, embedding_publisher):
    idx = x[:, 0].astype(jnp.int32)
    tableT = embedding_publisher.T
    mesh = plsc.VectorSubcoreMesh(core_axis_name="c", subcore_axis_name="s")
    run = functools.partial(
        pl.kernel,
        mesh=mesh,
        out_type=jax.ShapeDtypeStruct((_BATCH * _DIM,), jnp.float32),
        scratch_types=[
            pltpu.VMEM((_B_PER_W,), jnp.int32),
            pltpu.VMEM((_B_PER_W,), jnp.int32),
            pltpu.VMEM((_B_PER_W,), jnp.int32),
            pltpu.VMEM((_CHUNK, _DIM, _LANES), jnp.float32),
            pltpu.VMEM((_B_PER_W * _DIM,), jnp.float32),
            pltpu.SemaphoreType.DMA,
        ],
        compiler_params=pltpu.CompilerParams(needs_layout_passes=False),
    )(_gather_body)
    out_flat = run(idx, tableT)
    out = (out_flat.reshape(_DIM // 8, _NTC, 8, _LANES)
           .transpose(0, 2, 1, 3)
           .reshape(_DIM, _BATCH)
           .T)
    return out


# speedup vs baseline: 1.0636x; 1.0062x over previous
import functools

import jax
import jax.numpy as jnp
from jax import lax
from jax.experimental import pallas as pl
from jax.experimental.pallas import tpu as pltpu
from jax.experimental.pallas import tpu_sc as plsc

_BATCH = 16384
_DIM = 32
_LANES = 128
_NUM_CORES = 2
_NUM_SUBCORES = 16
_NUM_WORKERS = _NUM_CORES * _NUM_SUBCORES
_B_PER_W = _BATCH // _NUM_WORKERS
_CHUNK = 16
_NCHUNK = _B_PER_W // _CHUNK
_TC_PER_W = _B_PER_W // _LANES
_NTC = _BATCH // _LANES


def _gather_body(idx_hbm, tableT_hbm, out_hbm, idx_v, q_v, r_v, blocks_v,
                 out_v, sem):
    wid = lax.axis_index("s") * _NUM_CORES + lax.axis_index("c")
    base = wid * _B_PER_W
    pltpu.sync_copy(idx_hbm.at[pl.ds(base, _B_PER_W)], idx_v)

    @plsc.parallel_loop(0, _B_PER_W, 16)
    def _(i):
        v = idx_v[pl.ds(i, 16)]
        q_v[pl.ds(i, 16)] = v & jnp.int32(~(_LANES - 1))
        r_v[pl.ds(i, 16)] = v & jnp.int32(_LANES - 1)

    lanes = lax.iota(jnp.int32, 16)

    def chunk_body(g, carry):
        k0 = g * _CHUNK
        qv = q_v[pl.ds(k0, 16)]
        copies = []
        for j in range(_CHUNK):
            copies.append(pltpu.make_async_copy(
                tableT_hbm.at[
                    :, pl.ds(pl.multiple_of(qv[j], _LANES), _LANES)],
                blocks_v.at[j],
                sem,
            ))
        for c in copies:
            c.start()
        for c in copies:
            c.wait()

        rv = r_v[pl.ds(k0, 16)]
        tcl = k0 // _LANES
        kin = k0 % _LANES
        for c in range(_DIM):
            vals = plsc.load_gather(
                blocks_v, [lanes, jnp.full((16,), c, jnp.int32), rv])
            pos = ((c // 8) * _TC_PER_W + tcl) * 1024 + (c % 8) * 128 + kin
            out_v[pl.ds(pos, 16)] = vals
        return carry

    lax.fori_loop(0, _NCHUNK, chunk_body, 0)

    for tr in range(_DIM // 8):
        for tcl in range(_TC_PER_W):
            tc = wid * _TC_PER_W + tcl
            pltpu.sync_copy(
                out_v.at[pl.ds((tr * _TC_PER_W + tcl) * 1024, 1024)],
                out_hbm.at[pl.ds((tr * _NTC + tc) * 1024, 1024)],
            )


@jax.jit
def kernel(x, embedding_publisher):
    idx = x[:, 0].astype(jnp.int32)
    tableT = embedding_publisher.T
    mesh = plsc.VectorSubcoreMesh(core_axis_name="c", subcore_axis_name="s")
    run = functools.partial(
        pl.kernel,
        mesh=mesh,
        out_type=jax.ShapeDtypeStruct((_BATCH * _DIM,), jnp.float32),
        scratch_types=[
            pltpu.VMEM((_B_PER_W,), jnp.int32),
            pltpu.VMEM((_B_PER_W,), jnp.int32),
            pltpu.VMEM((_B_PER_W,), jnp.int32),
            pltpu.VMEM((_CHUNK, _DIM, _LANES), jnp.float32),
            pltpu.VMEM((_B_PER_W * _DIM,), jnp.float32),
            pltpu.SemaphoreType.DMA,
        ],
        compiler_params=pltpu.CompilerParams(needs_layout_passes=False),
    )(_gather_body)
    out_flat = run(idx, tableT)
    out = (out_flat.reshape(_DIM // 8, _NTC, 8, _LANES)
           .transpose(0, 2, 1, 3)
           .reshape(_DIM, _BATCH)
           .T)
    return out
